# baseline (speedup 1.0000x reference)
import math

import jax
import jax.numpy as jnp
from jax.experimental import pallas as pl
from jax.experimental.pallas import tpu as pltpu

HIDDEN = 256
IMAGE_SIZE = 1008
GRID = 72
B = 32
NPTS = 65
NSEM = 8
PATB = 1
TWO_PI = 2.0 * math.pi


def _body(pts_ref, lab_ref, pos_ref, tab_ref, nap_ref, nm_ref,
          sparse_ref, dense_ref, pat_ref, sp_ref, sem):
    pat_ref[...] = jnp.broadcast_to(nm_ref[...].reshape(1, 1, 1, HIDDEN),
                                    (PATB, GRID, GRID, HIDDEN))

    copies = [
        pltpu.make_async_copy(pat_ref,
                              dense_ref.at[pl.ds(g * PATB, PATB)],
                              sem.at[g % NSEM])
        for g in range(B // PATB)
    ]
    for c in copies:
        c.start()

    pts = pts_ref[...]
    px = pts[..., 0:1] + 0.5
    py = pts[..., 1:2] + 0.5
    labels = lab_ref[...]
    inv = 1.0 / IMAGE_SIZE
    bf = jnp.bfloat16
    cx = (2.0 * (px * inv) - 1.0).astype(bf).astype(jnp.float32)
    cy = (2.0 * (py * inv) - 1.0).astype(bf).astype(jnp.float32)
    p0 = pos_ref[0:1, :].reshape(1, 1, HIDDEN // 2)
    p1 = pos_ref[1:2, :].reshape(1, 1, HIDDEN // 2)
    p0 = p0.astype(bf).astype(jnp.float32)
    p1 = p1.astype(bf).astype(jnp.float32)
    c = TWO_PI * (cx * p0 + cy * p1)
    pe = jnp.concatenate([jnp.sin(c), jnp.cos(c)], axis=-1)
    nap = nap_ref[...].reshape(1, 1, HIDDEN)
    pe = jnp.where(labels == -1, nap, pe)
    pe = pe * (labels != -10).astype(pe.dtype)
    lc = jnp.maximum(labels, 0)
    e0 = tab_ref[0:1, :].reshape(1, 1, HIDDEN)
    e1 = tab_ref[1:2, :].reshape(1, 1, HIDDEN)
    e2 = tab_ref[2:3, :].reshape(1, 1, HIDDEN)
    e3 = tab_ref[3:4, :].reshape(1, 1, HIDDEN)
    pemb = jnp.where(lc == 0, e0,
                     jnp.where(lc == 1, e1,
                               jnp.where(lc == 2, e2, e3)))
    is_pos = (labels >= 0).astype(pe.dtype)
    sp_ref[:, 0:NPTS - 1, :] = pe + pemb * is_pos
    sp_ref[:, NPTS - 1:NPTS, :] = jnp.broadcast_to(nap, (B, 1, HIDDEN))
    c_sp = pltpu.make_async_copy(sp_ref, sparse_ref, sem.at[NSEM - 1])
    c_sp.start()

    for c in copies:
        c.wait()
    c_sp.wait()


def kernel(input_points, input_labels, positional_embedding, point_embed,
           not_a_point_embed, no_mask_embed):
    labels3 = input_labels[..., None]

    sparse, dense_hwc = pl.pallas_call(
        _body,
        in_specs=[
            pl.BlockSpec((B, NPTS - 1, 2), lambda: (0, 0, 0)),
            pl.BlockSpec((B, NPTS - 1, 1), lambda: (0, 0, 0)),
            pl.BlockSpec((2, HIDDEN // 2), lambda: (0, 0)),
            pl.BlockSpec((4, HIDDEN), lambda: (0, 0)),
            pl.BlockSpec((1, HIDDEN), lambda: (0, 0)),
            pl.BlockSpec((1, HIDDEN), lambda: (0, 0)),
        ],
        out_specs=[
            pl.BlockSpec(memory_space=pltpu.MemorySpace.HBM),
            pl.BlockSpec(memory_space=pltpu.MemorySpace.HBM),
        ],
        out_shape=[
            jax.ShapeDtypeStruct((B, NPTS, HIDDEN), jnp.float32),
            jax.ShapeDtypeStruct((B, GRID, GRID, HIDDEN), jnp.float32),
        ],
        scratch_shapes=[
            pltpu.MemorySpace.VMEM((PATB, GRID, GRID, HIDDEN), jnp.float32),
            pltpu.MemorySpace.VMEM((B, NPTS, HIDDEN), jnp.float32),
            pltpu.SemaphoreType.DMA((NSEM,)),
        ],
    )(input_points, labels3, positional_embedding, point_embed,
      not_a_point_embed, no_mask_embed)
    dense = dense_hwc.transpose(0, 3, 1, 2)
    return sparse, dense


# speedup vs baseline: 1.0431x; 1.0431x over previous
import math

import jax
import jax.numpy as jnp
from jax.experimental import pallas as pl
from jax.experimental.pallas import tpu as pltpu

HIDDEN = 256
IMAGE_SIZE = 1008
GRID = 72
B = 32
NPTS = 65
NSEM = 8
PATB = 1
TWO_PI = 2.0 * math.pi


def _body(px_ref, py_ref, lab_ref, pos_ref, tab_ref, nap_ref, nm_ref,
          sparse_ref, dense_ref, pat_ref, sp_ref, sem):
    pat_ref[...] = jnp.broadcast_to(nm_ref[...].reshape(1, 1, 1, HIDDEN),
                                    (PATB, GRID, GRID, HIDDEN))

    copies = [
        pltpu.make_async_copy(pat_ref,
                              dense_ref.at[pl.ds(g * PATB, PATB)],
                              sem.at[g % NSEM])
        for g in range(B // PATB)
    ]
    for c in copies:
        c.start()

    px = px_ref[...]
    py = py_ref[...]
    labels = lab_ref[...]
    inv = 1.0 / IMAGE_SIZE
    bf = jnp.bfloat16
    cx = (2.0 * (px * inv) - 1.0).astype(bf).astype(jnp.float32)
    cy = (2.0 * (py * inv) - 1.0).astype(bf).astype(jnp.float32)
    p0 = pos_ref[0:1, :].reshape(1, 1, HIDDEN // 2)
    p1 = pos_ref[1:2, :].reshape(1, 1, HIDDEN // 2)
    p0 = p0.astype(bf).astype(jnp.float32)
    p1 = p1.astype(bf).astype(jnp.float32)
    c = TWO_PI * (cx * p0 + cy * p1)
    pe = jnp.concatenate([jnp.sin(c), jnp.cos(c)], axis=-1)
    nap = nap_ref[...].reshape(1, 1, HIDDEN)
    pe = jnp.where(labels == -1, nap, pe)
    pe = pe * (labels != -10).astype(pe.dtype)
    lc = jnp.maximum(labels, 0)
    e0 = tab_ref[0:1, :].reshape(1, 1, HIDDEN)
    e1 = tab_ref[1:2, :].reshape(1, 1, HIDDEN)
    e2 = tab_ref[2:3, :].reshape(1, 1, HIDDEN)
    e3 = tab_ref[3:4, :].reshape(1, 1, HIDDEN)
    pemb = jnp.where(lc == 0, e0,
                     jnp.where(lc == 1, e1,
                               jnp.where(lc == 2, e2, e3)))
    is_pos = (labels >= 0).astype(pe.dtype)
    sp_ref[...] = pe + pemb * is_pos
    c_sp = pltpu.make_async_copy(sp_ref, sparse_ref, sem.at[NSEM - 1])
    c_sp.start()

    for c in copies:
        c.wait()
    c_sp.wait()


def kernel(input_points, input_labels, positional_embedding, point_embed,
           not_a_point_embed, no_mask_embed):
    pts = input_points + 0.5
    pts = jnp.concatenate([pts, jnp.zeros((B, 1, 2), pts.dtype)], axis=1)
    px = pts[..., 0:1]
    py = pts[..., 1:2]
    labels = jnp.concatenate(
        [input_labels, -jnp.ones((B, 1), input_labels.dtype)],
        axis=1)[..., None]

    sparse, dense_hwc = pl.pallas_call(
        _body,
        in_specs=[
            pl.BlockSpec((B, NPTS, 1), lambda: (0, 0, 0)),
            pl.BlockSpec((B, NPTS, 1), lambda: (0, 0, 0)),
            pl.BlockSpec((B, NPTS, 1), lambda: (0, 0, 0)),
            pl.BlockSpec((2, HIDDEN // 2), lambda: (0, 0)),
            pl.BlockSpec((4, HIDDEN), lambda: (0, 0)),
            pl.BlockSpec((1, HIDDEN), lambda: (0, 0)),
            pl.BlockSpec((1, HIDDEN), lambda: (0, 0)),
        ],
        out_specs=[
            pl.BlockSpec(memory_space=pltpu.MemorySpace.HBM),
            pl.BlockSpec(memory_space=pltpu.MemorySpace.HBM),
        ],
        out_shape=[
            jax.ShapeDtypeStruct((B, NPTS, HIDDEN), jnp.float32),
            jax.ShapeDtypeStruct((B, GRID, GRID, HIDDEN), jnp.float32),
        ],
        scratch_shapes=[
            pltpu.MemorySpace.VMEM((PATB, GRID, GRID, HIDDEN), jnp.float32),
            pltpu.MemorySpace.VMEM((B, NPTS, HIDDEN), jnp.float32),
            pltpu.SemaphoreType.DMA((NSEM,)),
        ],
    )(px, py, labels, positional_embedding, point_embed,
      not_a_point_embed, no_mask_embed)
    dense = dense_hwc.transpose(0, 3, 1, 2)
    return sparse, dense
